# baseline (speedup 1.0000x reference)
import functools

import jax
import jax.numpy as jnp
from jax import lax
from jax.experimental import pallas as pl
from jax.experimental.pallas import tpu as pltpu
from jax.experimental.pallas import tpu_sc as plsc

_R2 = 0.15 * 0.15
CAP = 64
NQP = 10240
QBA = 512
LCA = 512
QBB = 512
CHQ = 64
QPW = 640
NLAT_EXT = 2056


def _mask_body(q_ref, latT_ref, pm_ref,
               words_ref, cnt_ref, acc_ref):
    lc = pl.program_id(2)
    nlc = pl.num_programs(2)

    @pl.when(lc == 0)
    def _():
        acc_ref[...] = jnp.zeros_like(acc_ref)

    q = q_ref[0]
    latT = latT_ref[...]
    ql = jax.lax.dot_general(q, latT, (((1,), (0,)), ((), ())),
                             precision=jax.lax.Precision.HIGHEST,
                             preferred_element_type=jnp.float32)
    qn = jnp.sum(q * q, axis=1, keepdims=True)
    ln = jnp.sum(latT * latT, axis=0, keepdims=True)
    d2 = (qn + ln) - 2.0 * ql
    mask = (d2 <= _R2).astype(jnp.float32)

    wordsf = jax.lax.dot_general(mask, pm_ref[...], (((1,), (0,)), ((), ())),
                                 precision=jax.lax.Precision.HIGHEST,
                                 preferred_element_type=jnp.float32)
    words_ref[0, 0] = wordsf.astype(jnp.int32)

    acc_ref[...] += jnp.sum(mask, axis=1, keepdims=True)

    @pl.when(lc == nlc - 1)
    def _():
        cnt_ref[0] = acc_ref[...]


def _sc_body(words_hbm, ctab_hbm, fyext_hbm, gc_hbm, gfy_hbm,
             words_v, idx_v, c_tab, stag_c, stag_f, sem_in, sem_a, sem_f):
    c = lax.axis_index("c")
    s = lax.axis_index("s")
    b = c
    fy_off = b * NLAT_EXT
    sent = jnp.zeros((16,), jnp.int32) + (fy_off + 2048)
    lane16 = lax.iota(jnp.int32, 16) * 16

    pltpu.async_copy(ctab_hbm, c_tab, sem_a).wait()

    @pl.loop(0, QPW // CHQ)
    def _chunk(ch):
        q0 = s * QPW + ch * CHQ
        for l in range(4):
            pltpu.async_copy(words_hbm.at[b, l, pl.ds(q0, CHQ)],
                             words_v.at[l], sem_in).wait()

        @pl.loop(0, CHQ)
        def _query(qi):
            qrow = qi * CAP
            for t in range(CAP // 16):
                idx_v[pl.ds(qrow + t * 16, 16)] = sent

            off = jnp.int32(0)
            for v in range(8):
                w = words_v[v // 2, qi, pl.ds((v % 2) * 16, 16)]
                base = lane16 + (v * 256 + fy_off)

                def _cond(carry):
                    wc, _ = carry
                    return jnp.any(wc != 0)

                def _body(carry):
                    wc, offc = carry
                    m = wc != 0
                    lsb = wc & (0 - wc)
                    f32b = lax.bitcast_convert_type(
                        lsb.astype(jnp.float32), jnp.int32)
                    bit = lax.shift_right_logical(f32b, 23) - 127
                    gi = base + bit
                    offcl = jnp.minimum(offc, CAP - 16)
                    plsc.store_compressed(
                        idx_v.at[pl.ds(qrow + offcl, 16)], gi, mask=m)
                    nxt = offcl + jnp.sum(m.astype(jnp.int32))
                    return wc ^ lsb, nxt

                w, off = lax.while_loop(_cond, _body, (w, off))

        gbase = ((b * NQP + q0) * CAP)
        @pl.loop(0, CHQ * CAP // 256)
        def _jj(jj):
            copies = []
            for t in range(2):
                j = jj * 2 + t
                copies.append(pltpu.async_copy(
                    fyext_hbm.at[idx_v.at[pl.ds(j * 128, 128)]],
                    stag_f.at[pl.ds(t * 128, 128)], sem_f))
            for g in range(16):
                raw = idx_v[pl.ds(jj * 256 + g * 16, 16)] - fy_off
                for cc in range(4):
                    vals = plsc.load_gather(c_tab, [raw + cc * NLAT_EXT])
                    stag_c[cc, pl.ds(g * 16, 16)] = vals
            for cp in copies:
                cp.wait()
            pltpu.sync_copy(stag_c,
                            gc_hbm.at[:, pl.ds(gbase + jj * 256, 256)])
            pltpu.sync_copy(stag_f, gfy_hbm.at[pl.ds(gbase + jj * 256, 256)])


def _edge_body(q_ref, cnt_ref, gc_ref, gfy_ref, w0_ref, w04_ref, b0_ref,
               w1_ref, b1_ref, p0_ref, pb0_ref, p1_ref, pb1_ref, out_ref):
    q = q_ref[...]
    bq = jax.lax.dot_general(q, w0_ref[...][3:], (((1,), (0,)), ((), ())),
                             preferred_element_type=jnp.float32)
    a = jax.lax.dot_general(gc_ref[...], w04_ref[...], (((0,), (0,)), ((), ())),
                            preferred_element_type=jnp.float32)
    a = (a + b0_ref[...]).reshape(QBB, CAP, 32).astype(jnp.bfloat16)
    h = a + bq.astype(jnp.bfloat16)[:, None, :]
    h = jax.nn.gelu(h).reshape(QBB * CAP, 32)
    k = jax.lax.dot_general(h, w1_ref[...].astype(jnp.bfloat16),
                            (((1,), (0,)), ((), ())),
                            preferred_element_type=jnp.float32)
    k = k + b1_ref[...]
    msg = (k * gfy_ref[...]).reshape(QBB, CAP, 128)
    agg = jnp.sum(msg, axis=1)
    agg = agg / jnp.maximum(cnt_ref[...], 1.0)
    h2 = jax.lax.dot_general(agg, p0_ref[...], (((1,), (0,)), ((), ())),
                             preferred_element_type=jnp.float32)
    h2 = jax.nn.gelu(h2 + pb0_ref[...])
    out = jax.lax.dot_general(h2, p1_ref[...], (((1,), (0,)), ((), ())),
                              preferred_element_type=jnp.float32)
    out_ref[...] = out + pb1_ref[...]


def _run_sc(words, c_tab, fy_ext, B, ch):
    mesh = plsc.VectorSubcoreMesh(core_axis_name="c", subcore_axis_name="s")
    import dataclasses
    cp = pltpu.CompilerParams()
    if "needs_layout_passes" in pltpu.CompilerParams.__dataclass_fields__:
        cp = dataclasses.replace(cp, needs_layout_passes=False)
    sck = pl.kernel(
        _sc_body,
        mesh=mesh,
        compiler_params=cp,
        out_type=[
            jax.ShapeDtypeStruct((4, B * NQP * CAP), jnp.float32),
            jax.ShapeDtypeStruct((B * NQP * CAP, ch), jnp.float32),
        ],
        scratch_types=[
            pltpu.VMEM((4, CHQ, 32), jnp.int32),
            pltpu.VMEM((CHQ * CAP,), jnp.int32),
            pltpu.VMEM((4 * NLAT_EXT,), jnp.float32),
            pltpu.VMEM((4, 256), jnp.float32),
            pltpu.VMEM((256, ch), jnp.float32),
            pltpu.SemaphoreType.DMA,
            pltpu.SemaphoreType.DMA,
            pltpu.SemaphoreType.DMA,
        ],
    )
    return sck(words, c_tab, fy_ext)


def kernel(latent_tokens_coord, rndata, query_coord, W0, b0, W1, b1,
           P0, pb0, P1, pb1):
    B, Nq, cd = query_coord.shape
    Nl, _ = latent_tokens_coord.shape
    ch = rndata.shape[-1]
    nlc = Nl // LCA

    qpad = jnp.concatenate(
        [query_coord, jnp.full((B, NQP - Nq, cd), 2.0, jnp.float32)], axis=1)
    latT = latent_tokens_coord.T
    b0c = b0.reshape(1, -1)
    b1c = b1.reshape(1, -1)
    pb0c = pb0.reshape(1, -1)
    pb1c = pb1.reshape(1, -1)

    i = jnp.arange(LCA)
    pm = jnp.where(i[:, None] // 16 == jnp.arange(32)[None, :],
                   (2.0 ** (i % 16).astype(jnp.float32))[:, None],
                   0.0).astype(jnp.float32)

    words, cnt = pl.pallas_call(
        _mask_body,
        grid=(B, NQP // QBA, nlc),
        in_specs=[
            pl.BlockSpec((1, QBA, cd), lambda bb, qb, lc: (bb, qb, 0)),
            pl.BlockSpec((cd, LCA), lambda bb, qb, lc: (0, lc)),
            pl.BlockSpec((LCA, 32), lambda bb, qb, lc: (0, 0)),
        ],
        out_specs=[
            pl.BlockSpec((1, 1, QBA, 32), lambda bb, qb, lc: (bb, lc, qb, 0)),
            pl.BlockSpec((1, QBA, 1), lambda bb, qb, lc: (bb, qb, 0)),
        ],
        out_shape=[
            jax.ShapeDtypeStruct((B, nlc, NQP, 32), jnp.int32),
            jax.ShapeDtypeStruct((B, NQP, 1), jnp.float32),
        ],
        scratch_shapes=[pltpu.VMEM((QBA, 1), jnp.float32)],
        compiler_params=pltpu.CompilerParams(
            dimension_semantics=("parallel", "parallel", "arbitrary")),
    )(qpad, latT, pm)

    c_tab = jnp.zeros((4, NLAT_EXT), jnp.float32)
    c_tab = c_tab.at[:cd, :Nl].set(latT).reshape(-1)
    w04 = jnp.concatenate([W0[:cd], jnp.zeros((1, W0.shape[1]), jnp.float32)],
                          axis=0)
    fy_ext = jnp.concatenate(
        [rndata, jnp.zeros((B, NLAT_EXT - Nl, ch), jnp.float32)], axis=1)
    fy_ext = fy_ext.reshape(B * NLAT_EXT, ch)

    gc, gfy = _run_sc(words, c_tab, fy_ext, B, ch)

    out = pl.pallas_call(
        _edge_body,
        grid=(B * NQP // QBB,),
        in_specs=[
            pl.BlockSpec((QBB, cd), lambda g: (g, 0)),
            pl.BlockSpec((QBB, 1), lambda g: (g, 0)),
            pl.BlockSpec((4, QBB * CAP), lambda g: (0, g)),
            pl.BlockSpec((QBB * CAP, ch), lambda g: (g, 0)),
            pl.BlockSpec(W0.shape, lambda g: (0, 0)),
            pl.BlockSpec((4, W0.shape[1]), lambda g: (0, 0)),
            pl.BlockSpec((1, b0.shape[0]), lambda g: (0, 0)),
            pl.BlockSpec(W1.shape, lambda g: (0, 0)),
            pl.BlockSpec((1, b1.shape[0]), lambda g: (0, 0)),
            pl.BlockSpec(P0.shape, lambda g: (0, 0)),
            pl.BlockSpec((1, pb0.shape[0]), lambda g: (0, 0)),
            pl.BlockSpec(P1.shape, lambda g: (0, 0)),
            pl.BlockSpec((1, pb1.shape[0]), lambda g: (0, 0)),
        ],
        out_specs=pl.BlockSpec((QBB, ch), lambda g: (g, 0)),
        out_shape=jax.ShapeDtypeStruct((B * NQP, ch), jnp.float32),
        compiler_params=pltpu.CompilerParams(
            dimension_semantics=("parallel",)),
    )(qpad.reshape(B * NQP, cd), cnt.reshape(B * NQP, 1), gc, gfy,
      W0, w04, b0c, W1, b1c, P0, pb0c, P1, pb1c)

    return out.reshape(B, NQP, ch)[:, :Nq]


# speedup vs baseline: 9.1338x; 9.1338x over previous
import functools

import jax
import jax.numpy as jnp
from jax.experimental import pallas as pl
from jax.experimental.pallas import tpu as pltpu

_R2 = 0.15 * 0.15


def _body(q_ref, latT_ref, fy_ref, w0_ref, b0_ref, w1_ref, b1_ref,
          p0_ref, pb0_ref, p1_ref, pb1_ref, out_ref,
          res_ref, sumf_ref, cnt_ref, *, qb_size, lc_size):
    lc = pl.program_id(2)
    nlc = pl.num_programs(2)

    @pl.when(lc == 0)
    def _():
        res_ref[...] = jnp.zeros_like(res_ref)
        sumf_ref[...] = jnp.zeros_like(sumf_ref)
        cnt_ref[...] = jnp.zeros_like(cnt_ref)

    q = q_ref[0]
    latT = latT_ref[...]

    ql = jax.lax.dot_general(q, latT, (((1,), (0,)), ((), ())),
                             precision=jax.lax.Precision.HIGHEST,
                             preferred_element_type=jnp.float32)
    qn = jnp.sum(q * q, axis=1, keepdims=True)
    ln = jnp.sum(latT * latT, axis=0, keepdims=True)
    d2 = (qn + ln) - 2.0 * ql
    mask = d2 <= _R2

    w0 = w0_ref[...]
    a_mat = w0[:3]
    b_mat = w0[3:]
    bq = jax.lax.dot_general(q, b_mat, (((1,), (0,)), ((), ())),
                             preferred_element_type=jnp.float32)
    aT = jax.lax.dot_general(a_mat, latT, (((0,), (0,)), ((), ())),
                             preferred_element_type=jnp.float32)
    aT = aT + b0_ref[...]
    h = (bq.astype(jnp.bfloat16)[:, :, None]
         + aT.astype(jnp.bfloat16)[None, :, :])
    h = jax.nn.gelu(h)
    mh = jnp.where(mask[:, None, :], h, jnp.bfloat16(0.0))

    fy = fy_ref[0]
    mh2 = mh.reshape(qb_size * 32, lc_size)
    res_ref[...] += jax.lax.dot_general(
        mh2, fy, (((1,), (0,)), ((), ())),
        preferred_element_type=jnp.float32).reshape(qb_size, 32, 128)
    sumf_ref[...] += jax.lax.dot_general(
        mask.astype(jnp.bfloat16), fy, (((1,), (0,)), ((), ())),
        preferred_element_type=jnp.float32)
    cnt_ref[...] += jnp.sum(mask.astype(jnp.float32), axis=1, keepdims=True)

    @pl.when(lc == nlc - 1)
    def _():
        w1 = w1_ref[...]
        agg = jnp.sum(res_ref[...] * w1[None, :, :], axis=1)
        agg = agg + b1_ref[...] * sumf_ref[...]
        agg = agg / jnp.maximum(cnt_ref[...], 1.0)
        h2 = jax.lax.dot_general(agg, p0_ref[...], (((1,), (0,)), ((), ())),
                                 preferred_element_type=jnp.float32)
        h2 = jax.nn.gelu(h2 + pb0_ref[...])
        out = jax.lax.dot_general(h2, p1_ref[...], (((1,), (0,)), ((), ())),
                                  preferred_element_type=jnp.float32)
        out_ref[0] = out + pb1_ref[...]


def kernel(latent_tokens_coord, rndata, query_coord, W0, b0, W1, b1,
           P0, pb0, P1, pb1):
    B, Nq, cd = query_coord.shape
    Nl, _ = latent_tokens_coord.shape
    ch = rndata.shape[-1]

    QB = 400
    LC = 256
    nqb = Nq // QB
    nlc = Nl // LC

    latT = latent_tokens_coord.T
    fy = rndata.astype(jnp.bfloat16)
    b0c = b0.reshape(-1, 1)
    b1c = b1.reshape(1, -1)
    pb0c = pb0.reshape(1, -1)
    pb1c = pb1.reshape(1, -1)

    body = functools.partial(_body, qb_size=QB, lc_size=LC)

    out = pl.pallas_call(
        body,
        grid=(B, nqb, nlc),
        in_specs=[
            pl.BlockSpec((1, QB, cd), lambda b, qb, lc: (b, qb, 0)),
            pl.BlockSpec((cd, LC), lambda b, qb, lc: (0, lc)),
            pl.BlockSpec((1, LC, ch), lambda b, qb, lc: (b, lc, 0)),
            pl.BlockSpec(W0.shape, lambda b, qb, lc: (0, 0)),
            pl.BlockSpec((b0.shape[0], 1), lambda b, qb, lc: (0, 0)),
            pl.BlockSpec(W1.shape, lambda b, qb, lc: (0, 0)),
            pl.BlockSpec((1, b1.shape[0]), lambda b, qb, lc: (0, 0)),
            pl.BlockSpec(P0.shape, lambda b, qb, lc: (0, 0)),
            pl.BlockSpec((1, pb0.shape[0]), lambda b, qb, lc: (0, 0)),
            pl.BlockSpec(P1.shape, lambda b, qb, lc: (0, 0)),
            pl.BlockSpec((1, pb1.shape[0]), lambda b, qb, lc: (0, 0)),
        ],
        out_specs=pl.BlockSpec((1, QB, ch), lambda b, qb, lc: (b, qb, 0)),
        out_shape=jax.ShapeDtypeStruct((B, Nq, ch), jnp.float32),
        scratch_shapes=[
            pltpu.VMEM((QB, 32, ch), jnp.float32),
            pltpu.VMEM((QB, ch), jnp.float32),
            pltpu.VMEM((QB, 1), jnp.float32),
        ],
        compiler_params=pltpu.CompilerParams(
            dimension_semantics=("parallel", "parallel", "arbitrary"),
        ),
    )(query_coord, latT, fy, W0, b0c, W1, b1c, P0, pb0c, P1, pb1c)
    return out
